# baseline (speedup 1.0000x reference)
import functools

import jax
import jax.numpy as jnp
from jax import lax
from jax.experimental import pallas as pl
from jax.experimental.pallas import tpu as pltpu
from jax.experimental.pallas import tpu_sc as plsc

_LANES = 16


@functools.lru_cache(maxsize=None)
def _make_gather(batch, dim):
    info = plsc.get_sparse_core_info()
    nc, ns = info.num_cores, info.num_subcores
    nw = nc * ns
    b_per_w = batch // nw
    n_groups = b_per_w // _LANES
    rows_per_line = 128 // dim
    lines_out = b_per_w // rows_per_line
    mesh = plsc.VectorSubcoreMesh(core_axis_name="c", subcore_axis_name="s")

    @functools.partial(
        pl.kernel,
        mesh=mesh,
        out_type=jax.ShapeDtypeStruct((batch // rows_per_line, 128), jnp.float32),
        scratch_types=[
            pltpu.VMEM((b_per_w,), jnp.int32),
            pltpu.VMEM((b_per_w, dim), jnp.float32),
            pltpu.VMEM((lines_out, 128), jnp.float32),
            pltpu.SemaphoreType.DMA,
        ],
    )
    def gather_kernel(idx_hbm, table_hbm, out_hbm, idx_v, rows_v, pack_v, sem):
        wid = lax.axis_index("s") * nc + lax.axis_index("c")
        base = wid * b_per_w
        pltpu.sync_copy(idx_hbm.at[pl.ds(base, b_per_w)], idx_v)

        def issue(g, carry):
            v = idx_v[pl.ds(g * _LANES, _LANES)]
            for lane in range(_LANES):
                pltpu.async_copy(
                    table_hbm.at[pl.ds(v[lane], 1)],
                    rows_v.at[pl.ds(g * _LANES + lane, 1)],
                    sem,
                )
            return carry

        lax.fori_loop(0, n_groups, issue, 0)
        pltpu.make_async_copy(
            table_hbm.at[pl.ds(0, b_per_w)], rows_v, sem
        ).wait()

        def repack(g, carry):
            for lane in range(_LANES):
                b = g * _LANES + lane
                line = g * (_LANES // rows_per_line) + lane // rows_per_line
                col = (lane % rows_per_line) * dim
                for h in range(dim // _LANES):
                    pack_v[line, pl.ds(col + h * _LANES, _LANES)] = rows_v[
                        b, pl.ds(h * _LANES, _LANES)
                    ]
            return carry

        lax.fori_loop(0, n_groups, repack, 0)
        pltpu.sync_copy(pack_v, out_hbm.at[pl.ds(wid * lines_out, lines_out)])

    return gather_kernel


def kernel(indices, table):
    batch = indices.shape[0]
    dim = table.shape[1]
    out_lines = _make_gather(batch, dim)(indices, table)
    return out_lines.reshape(batch, dim)


# speedup vs baseline: 1.7073x; 1.7073x over previous
import functools

import jax
import jax.numpy as jnp
from jax import lax
from jax.experimental import pallas as pl
from jax.experimental.pallas import tpu as pltpu
from jax.experimental.pallas import tpu_sc as plsc

_LANES = 16


@functools.lru_cache(maxsize=None)
def _make_gather(batch, dim):
    info = plsc.get_sparse_core_info()
    nc, ns = info.num_cores, info.num_subcores
    nw = nc * ns
    b_per_w = batch // nw
    n_groups = b_per_w // _LANES
    mesh = plsc.VectorSubcoreMesh(core_axis_name="c", subcore_axis_name="s")

    @functools.partial(
        pl.kernel,
        mesh=mesh,
        out_type=jax.ShapeDtypeStruct((batch, dim), jnp.float32),
        scratch_types=[
            pltpu.VMEM((b_per_w,), jnp.int32),
            pltpu.VMEM((b_per_w, dim), jnp.float32),
            pltpu.SemaphoreType.DMA,
        ],
    )
    def gather_kernel(idx_hbm, table3_hbm, out_hbm, idx_v, rows_v, sem):
        wid = lax.axis_index("s") * nc + lax.axis_index("c")
        base = wid * b_per_w
        pltpu.sync_copy(idx_hbm.at[pl.ds(base, b_per_w)], idx_v)

        def issue(g, carry):
            v = idx_v[pl.ds(g * _LANES, _LANES)]
            a_vec = lax.shift_right_logical(v, 3)
            r_vec = lax.bitwise_and(v, 7)
            for lane in range(_LANES):
                pltpu.async_copy(
                    table3_hbm.at[a_vec[lane], pl.ds(r_vec[lane], 1)],
                    rows_v.at[pl.ds(g * _LANES + lane, 1)],
                    sem,
                )
            return carry

        lax.fori_loop(0, n_groups, issue, 0)
        pltpu.make_async_copy(
            table3_hbm.at[pl.ds(0, b_per_w // 8)],
            rows_v.reshape(b_per_w // 8, 8, dim),
            sem,
        ).wait()
        pltpu.sync_copy(rows_v, out_hbm.at[pl.ds(base, b_per_w)])

    return gather_kernel


def kernel(indices, table):
    batch = indices.shape[0]
    vocab, dim = table.shape
    table3 = table.reshape(vocab // 8, 8, dim)
    return _make_gather(batch, dim)(indices, table3)


# speedup vs baseline: 1.7091x; 1.0010x over previous
import functools

import jax
import jax.numpy as jnp
from jax import lax
from jax.experimental import pallas as pl
from jax.experimental.pallas import tpu as pltpu
from jax.experimental.pallas import tpu_sc as plsc

_LANES = 16
_CHUNKS = 4


@functools.lru_cache(maxsize=None)
def _make_gather(batch, dim):
    info = plsc.get_sparse_core_info()
    nc, ns = info.num_cores, info.num_subcores
    nw = nc * ns
    b_per_w = batch // nw
    b_per_c = b_per_w // _CHUNKS
    groups_per_c = b_per_c // _LANES
    mesh = plsc.VectorSubcoreMesh(core_axis_name="c", subcore_axis_name="s")

    @functools.partial(
        pl.kernel,
        mesh=mesh,
        out_type=jax.ShapeDtypeStruct((batch, dim), jnp.float32),
        scratch_types=[
            pltpu.VMEM((b_per_w,), jnp.int32),
            pltpu.VMEM((b_per_w, dim), jnp.float32),
        ]
        + [pltpu.SemaphoreType.DMA] * _CHUNKS
        + [pltpu.SemaphoreType.DMA],
    )
    def gather_kernel(idx_hbm, table3_hbm, out_hbm, idx_v, rows_v, *sems):
        gsems, wsem = sems[:_CHUNKS], sems[_CHUNKS]
        wid = lax.axis_index("s") * nc + lax.axis_index("c")
        base = wid * b_per_w
        pltpu.sync_copy(idx_hbm.at[pl.ds(base, b_per_w)], idx_v)

        for c in range(_CHUNKS):
            def issue(g, carry, c=c):
                b0 = c * b_per_c + g * _LANES
                v = idx_v[pl.ds(b0, _LANES)]
                a_vec = lax.shift_right_logical(v, 3)
                r_vec = lax.bitwise_and(v, 7)
                for lane in range(_LANES):
                    pltpu.async_copy(
                        table3_hbm.at[a_vec[lane], pl.ds(r_vec[lane], 1)],
                        rows_v.at[pl.ds(b0 + lane, 1)],
                        gsems[c],
                    )
                return carry

            lax.fori_loop(0, groups_per_c, issue, 0)

        for c in range(_CHUNKS):
            pltpu.make_async_copy(
                table3_hbm.at[pl.ds(0, b_per_c // 8)],
                rows_v.at[pl.ds(c * b_per_c, b_per_c)].reshape(
                    b_per_c // 8, 8, dim
                ),
                gsems[c],
            ).wait()
            pltpu.async_copy(
                rows_v.at[pl.ds(c * b_per_c, b_per_c)],
                out_hbm.at[pl.ds(base + c * b_per_c, b_per_c)],
                wsem,
            )
        pltpu.make_async_copy(
            table3_hbm.at[pl.ds(0, b_per_w // 8)],
            rows_v.reshape(b_per_w // 8, 8, dim),
            wsem,
        ).wait()

    return gather_kernel


def kernel(indices, table):
    batch = indices.shape[0]
    vocab, dim = table.shape
    table3 = table.reshape(vocab // 8, 8, dim)
    return _make_gather(batch, dim)(indices, table3)
